# Initial kernel scaffold; baseline (speedup 1.0000x reference)
import functools

import jax
import jax.numpy as jnp
from jax import lax
from jax.experimental import pallas as pl
from jax.experimental.pallas import tpu as pltpu
from jax.experimental.pallas import tpu_sc as plsc

NC = 2
NS = 16
LANES = 16
ZR = 25


def _sc_aggregate(x2, src, dst, n_nodes, n_edges, wc, with_deg):
    ept = n_edges // NS
    K = 80
    n_chunks = ept // K
    assert ept % K == 0 and n_nodes % NS == 0 and (n_nodes // NS) % ZR == 0
    rpt = n_nodes // NS

    mesh = plsc.VectorSubcoreMesh(core_axis_name="c", subcore_axis_name="s")

    out_type = [jax.ShapeDtypeStruct((NC * n_nodes, wc), jnp.float32)]
    scratch = [
        pltpu.VMEM_SHARED((n_nodes, wc), jnp.float32),
        pltpu.VMEM((K,), jnp.int32),
        pltpu.VMEM((K,), jnp.int32),
        pltpu.VMEM((K, wc), jnp.float32),
        pltpu.VMEM((ZR, wc), jnp.float32),
        pltpu.SemaphoreType.DMA,
    ]
    if with_deg:
        out_type.append(jax.ShapeDtypeStruct((NC * n_nodes, LANES), jnp.float32))
        scratch += [
            pltpu.VMEM_SHARED((n_nodes, LANES), jnp.float32),
            pltpu.VMEM((K, LANES), jnp.float32),
            pltpu.VMEM((ZR, LANES), jnp.float32),
        ]

    @functools.partial(pl.kernel, out_type=tuple(out_type), mesh=mesh,
                       scratch_types=tuple(scratch))
    def agg_kernel(*refs):
        if with_deg:
            (src_h, dst_h, x_h, out_h, deg_h,
             acc, sidx, didx, rows, zbuf, sem, dacc, ones_v, dzbuf) = refs
        else:
            (src_h, dst_h, x_h, out_h,
             acc, sidx, didx, rows, zbuf, sem) = refs
        c = lax.axis_index("c")
        s = lax.axis_index("s")

        zvec = jnp.zeros((LANES,), jnp.float32)

        def fill_zbuf(r, _):
            for k in range(wc // LANES):
                zbuf[r, pl.ds(k * LANES, LANES)] = zvec
            return 0
        lax.fori_loop(0, ZR, fill_zbuf, 0)

        def zero_acc(j, _):
            pltpu.sync_copy(zbuf, acc.at[pl.ds(s * rpt + j * ZR, ZR)])
            return 0
        lax.fori_loop(0, rpt // ZR, zero_acc, 0)

        if with_deg:
            ovec = jnp.full((LANES,), 1.0, jnp.float32)

            def fill_ones(r, _):
                ones_v[r, pl.ds(0, LANES)] = ovec
                return 0
            lax.fori_loop(0, K, fill_ones, 0)

            def fill_dzbuf(r, _):
                dzbuf[r, pl.ds(0, LANES)] = zvec
                return 0
            lax.fori_loop(0, ZR, fill_dzbuf, 0)

            def zero_deg(j, _):
                pltpu.sync_copy(dzbuf, dacc.at[pl.ds(s * rpt + j * ZR, ZR)])
                return 0
            lax.fori_loop(0, rpt // ZR, zero_deg, 0)

        plsc.subcore_barrier()

        e0 = s * ept
        off = c * n_nodes

        def chunk(i, _):
            base = e0 + i * K
            pltpu.sync_copy(src_h.at[pl.ds(base, K)], sidx)
            pltpu.sync_copy(dst_h.at[pl.ds(base, K)], didx)
            for j in range(K // LANES):
                sl = pl.ds(j * LANES, LANES)
                sidx[sl] = sidx[sl] + off
            pltpu.async_copy(x_h.at[sidx], rows, sem).wait()
            pltpu.sync_copy(rows, acc.at[didx], add=True)
            if with_deg:
                pltpu.sync_copy(ones_v, dacc.at[didx], add=True)
            return 0
        lax.fori_loop(0, n_chunks, chunk, 0)

        plsc.subcore_barrier()

        r0 = s * rpt
        pltpu.sync_copy(acc.at[pl.ds(r0, rpt)], out_h.at[pl.ds(off + r0, rpt)])
        if with_deg:
            pltpu.sync_copy(dacc.at[pl.ds(r0, rpt)],
                            deg_h.at[pl.ds(off + r0, rpt)])

    return agg_kernel(src, dst, x2)


def _tc_combine1(x, agg, deg, Ws, Wn, b):
    n, d = x.shape
    h = Ws.shape[1]
    R = 1000

    def body(x_r, agg_r, deg_r, Ws_r, Wn_r, b_r, out_r):
        inv = 1.0 / jnp.maximum(deg_r[...], 1.0)
        hn = agg_r[...] * inv
        acc = jnp.dot(x_r[...], Ws_r[...], preferred_element_type=jnp.float32)
        acc = acc + jnp.dot(hn, Wn_r[...], preferred_element_type=jnp.float32)
        out_r[...] = jnp.maximum(acc + b_r[...], 0.0)

    return pl.pallas_call(
        body,
        grid=(n // R,),
        in_specs=[
            pl.BlockSpec((R, d), lambda i: (i, 0)),
            pl.BlockSpec((R, d), lambda i: (i, 0)),
            pl.BlockSpec((R, 1), lambda i: (i, 0)),
            pl.BlockSpec((d, h), lambda i: (0, 0)),
            pl.BlockSpec((d, h), lambda i: (0, 0)),
            pl.BlockSpec((1, h), lambda i: (0, 0)),
        ],
        out_specs=pl.BlockSpec((R, h), lambda i: (i, 0)),
        out_shape=jax.ShapeDtypeStruct((n, h), jnp.float32),
    )(x, agg, deg, Ws, Wn, b.reshape(1, h))


def _tc_combine2(h1, agg, deg, Ws2, Wn2, b2, Ws3, Wn3, b3):
    n, h = h1.shape
    cdim = Ws3.shape[1]
    R = 1000

    def body(h1_r, agg_r, deg_r, Ws2_r, Wn2_r, b2_r, Ws3_r, Wn3_r, b3_r,
             p_r, q_r):
        inv = 1.0 / jnp.maximum(deg_r[...], 1.0)
        hn = agg_r[...] * inv
        acc = jnp.dot(h1_r[...], Ws2_r[...], preferred_element_type=jnp.float32)
        acc = acc + jnp.dot(hn, Wn2_r[...], preferred_element_type=jnp.float32)
        h2 = jnp.maximum(acc + b2_r[...], 0.0)
        p_r[...] = jnp.dot(h2, Wn3_r[...], preferred_element_type=jnp.float32)
        q_r[...] = jnp.dot(h2, Ws3_r[...],
                           preferred_element_type=jnp.float32) + b3_r[...]

    return pl.pallas_call(
        body,
        grid=(n // R,),
        in_specs=[
            pl.BlockSpec((R, h), lambda i: (i, 0)),
            pl.BlockSpec((R, h), lambda i: (i, 0)),
            pl.BlockSpec((R, 1), lambda i: (i, 0)),
            pl.BlockSpec((h, h), lambda i: (0, 0)),
            pl.BlockSpec((h, h), lambda i: (0, 0)),
            pl.BlockSpec((1, h), lambda i: (0, 0)),
            pl.BlockSpec((h, cdim), lambda i: (0, 0)),
            pl.BlockSpec((h, cdim), lambda i: (0, 0)),
            pl.BlockSpec((1, cdim), lambda i: (0, 0)),
        ],
        out_specs=[
            pl.BlockSpec((R, cdim), lambda i: (i, 0)),
            pl.BlockSpec((R, cdim), lambda i: (i, 0)),
        ],
        out_shape=[
            jax.ShapeDtypeStruct((n, cdim), jnp.float32),
            jax.ShapeDtypeStruct((n, cdim), jnp.float32),
        ],
    )(h1, agg, deg, Ws2, Wn2, b2.reshape(1, h), Ws3, Wn3, b3.reshape(1, cdim))


def _tc_final(q, agg, deg):
    n, cdim = q.shape
    R = 1000

    def body(q_r, agg_r, deg_r, out_r):
        inv = 1.0 / jnp.maximum(deg_r[...], 1.0)
        out_r[...] = q_r[...] + agg_r[...] * inv

    return pl.pallas_call(
        body,
        grid=(n // R,),
        in_specs=[
            pl.BlockSpec((R, cdim), lambda i: (i, 0)),
            pl.BlockSpec((R, cdim), lambda i: (i, 0)),
            pl.BlockSpec((R, 1), lambda i: (i, 0)),
        ],
        out_specs=pl.BlockSpec((R, cdim), lambda i: (i, 0)),
        out_shape=jax.ShapeDtypeStruct((n, cdim), jnp.float32),
    )(q, agg, deg)


def _split2(x):
    n, w = x.shape
    return x.reshape(n, NC, w // NC).transpose(1, 0, 2).reshape(NC * n, w // NC)


def _unsplit2(y, n, w):
    return y.reshape(NC, n, w // NC).transpose(1, 0, 2).reshape(n, w)


def kernel(features, edge_index, Ws1, Wn1, b1, Ws2, Wn2, b2, Ws3, Wn3, b3):
    n, d = features.shape
    h = Ws1.shape[1]
    cdim = Ws3.shape[1]
    e = edge_index.shape[1]
    src = edge_index[0]
    dst = edge_index[1]

    agg1_2, degf = _sc_aggregate(_split2(features), src, dst, n, e,
                                 d // NC, True)
    deg = degf[:n, 0:1]
    h1 = _tc_combine1(features, _unsplit2(agg1_2, n, d), deg, Ws1, Wn1, b1)

    agg2_2 = _sc_aggregate(_split2(h1), src, dst, n, e, h // NC, False)[0]
    p3, q3 = _tc_combine2(h1, _unsplit2(agg2_2, n, h), deg,
                          Ws2, Wn2, b2, Ws3, Wn3, b3)

    agg3_2 = _sc_aggregate(_split2(p3), src, dst, n, e, cdim // NC, False)[0]
    return _tc_final(q3, _unsplit2(agg3_2, n, cdim), deg)


# speedup vs baseline: 4.8777x; 4.8777x over previous
import functools

import jax
import jax.numpy as jnp
from jax import lax
from jax.experimental import pallas as pl
from jax.experimental.pallas import tpu as pltpu
from jax.experimental.pallas import tpu_sc as plsc

NC = 2
NS = 16
LANES = 16
ZR = 80


def _sc_aggregate(x2, src, dst, n_nodes, n_edges, wc, edge_split, with_deg):
    ept = n_edges // (NC * NS if edge_split else NS)
    K = 80
    n_chunks = ept // K
    assert ept % K == 0 and n_nodes % (NS * ZR) == 0
    rpt = n_nodes // NS

    mesh = plsc.VectorSubcoreMesh(core_axis_name="c", subcore_axis_name="s")

    out_type = [jax.ShapeDtypeStruct((NC * n_nodes, wc), jnp.float32)]
    scratch = [
        pltpu.VMEM_SHARED((n_nodes, wc), jnp.float32),
        pltpu.VMEM((K,), jnp.int32),
        pltpu.VMEM((K,), jnp.int32),
        pltpu.VMEM((K, wc), jnp.float32),
        pltpu.VMEM((ZR, wc), jnp.float32),
        pltpu.SemaphoreType.DMA,
    ]
    if with_deg:
        out_type.append(jax.ShapeDtypeStruct((NC * n_nodes, LANES), jnp.float32))
        scratch += [
            pltpu.VMEM_SHARED((n_nodes, LANES), jnp.float32),
            pltpu.VMEM((K, LANES), jnp.float32),
            pltpu.VMEM((ZR, LANES), jnp.float32),
        ]

    @functools.partial(
        pl.kernel, out_type=tuple(out_type), mesh=mesh,
        scratch_types=tuple(scratch),
        compiler_params=pltpu.CompilerParams(use_tc_tiling_on_sc=False))
    def agg_kernel(*refs):
        if with_deg:
            (src_h, dst_h, x_h, out_h, deg_h,
             acc, sidx, didx, rows, zbuf, sem, dacc, ones_v, dzbuf) = refs
        else:
            (src_h, dst_h, x_h, out_h,
             acc, sidx, didx, rows, zbuf, sem) = refs
        c = lax.axis_index("c")
        s = lax.axis_index("s")

        zvec = jnp.zeros((LANES,), jnp.float32)

        def fill_zbuf(r, _):
            for k in range(wc // LANES):
                zbuf[r, pl.ds(k * LANES, LANES)] = zvec
            return 0
        lax.fori_loop(0, ZR, fill_zbuf, 0)

        def zero_acc(j, _):
            pltpu.sync_copy(zbuf, acc.at[pl.ds(s * rpt + j * ZR, ZR)])
            return 0
        lax.fori_loop(0, rpt // ZR, zero_acc, 0)

        if with_deg:
            ovec = jnp.full((LANES,), 1.0, jnp.float32)

            def fill_ones(r, _):
                ones_v[r, pl.ds(0, LANES)] = ovec
                return 0
            lax.fori_loop(0, K, fill_ones, 0)

            def fill_dzbuf(r, _):
                dzbuf[r, pl.ds(0, LANES)] = zvec
                return 0
            lax.fori_loop(0, ZR, fill_dzbuf, 0)

            def zero_deg(j, _):
                pltpu.sync_copy(dzbuf, dacc.at[pl.ds(s * rpt + j * ZR, ZR)])
                return 0
            lax.fori_loop(0, rpt // ZR, zero_deg, 0)

        plsc.subcore_barrier()

        if edge_split:
            e0 = (c * NS + s) * ept
        else:
            e0 = s * ept
        off = c * n_nodes

        def chunk(i, _):
            base = e0 + i * K
            pltpu.sync_copy(src_h.at[pl.ds(base, K)], sidx)
            pltpu.sync_copy(dst_h.at[pl.ds(base, K)], didx)
            if not edge_split:
                for j in range(K // LANES):
                    sl = pl.ds(j * LANES, LANES)
                    sidx[sl] = sidx[sl] + off
            pltpu.async_copy(x_h.at[sidx], rows, sem).wait()
            pltpu.sync_copy(rows, acc.at[didx], add=True)
            if with_deg:
                pltpu.sync_copy(ones_v, dacc.at[didx], add=True)
            return 0
        lax.fori_loop(0, n_chunks, chunk, 0)

        plsc.subcore_barrier()

        r0 = s * rpt
        pltpu.sync_copy(acc.at[pl.ds(r0, rpt)], out_h.at[pl.ds(off + r0, rpt)])
        if with_deg:
            pltpu.sync_copy(dacc.at[pl.ds(r0, rpt)],
                            deg_h.at[pl.ds(off + r0, rpt)])

    return agg_kernel(src, dst, x2)


def _tc_combine1(x, agg_a, agg_b, deg2, Ws, Wn, b):
    n, d = x.shape
    h = Ws.shape[1]
    R = 1000

    def body(x_r, aa_r, ab_r, deg_r, Ws_r, Wn_r, b_r, out_r):
        deg = deg_r[:, 0:1] + deg_r[:, 1:2]
        inv = 1.0 / jnp.maximum(deg, 1.0)
        hn = (aa_r[...] + ab_r[...]) * inv
        acc = jnp.dot(x_r[...], Ws_r[...], preferred_element_type=jnp.float32)
        acc = acc + jnp.dot(hn, Wn_r[...], preferred_element_type=jnp.float32)
        out_r[...] = jnp.maximum(acc + b_r[...], 0.0)

    return pl.pallas_call(
        body,
        grid=(n // R,),
        in_specs=[
            pl.BlockSpec((R, d), lambda i: (i, 0)),
            pl.BlockSpec((R, d), lambda i: (i, 0)),
            pl.BlockSpec((R, d), lambda i: (i, 0)),
            pl.BlockSpec((R, 2), lambda i: (i, 0)),
            pl.BlockSpec((d, h), lambda i: (0, 0)),
            pl.BlockSpec((d, h), lambda i: (0, 0)),
            pl.BlockSpec((1, h), lambda i: (0, 0)),
        ],
        out_specs=pl.BlockSpec((R, h), lambda i: (i, 0)),
        out_shape=jax.ShapeDtypeStruct((n, h), jnp.float32),
    )(x, agg_a, agg_b, deg2, Ws, Wn, b.reshape(1, h))


def _tc_combine2(h1, agg, deg2, Ws2, Wn2, b2, Ws3, Wn3, b3):
    n, h = h1.shape
    cdim = Ws3.shape[1]
    R = 1000

    def body(h1_r, agg_r, deg_r, Ws2_r, Wn2_r, b2_r, Ws3_r, Wn3_r, b3_r,
             p_r, q_r):
        deg = deg_r[:, 0:1] + deg_r[:, 1:2]
        inv = 1.0 / jnp.maximum(deg, 1.0)
        hn = agg_r[...] * inv
        acc = jnp.dot(h1_r[...], Ws2_r[...], preferred_element_type=jnp.float32)
        acc = acc + jnp.dot(hn, Wn2_r[...], preferred_element_type=jnp.float32)
        h2 = jnp.maximum(acc + b2_r[...], 0.0)
        p_r[...] = jnp.dot(h2, Wn3_r[...], preferred_element_type=jnp.float32)
        q_r[...] = jnp.dot(h2, Ws3_r[...],
                           preferred_element_type=jnp.float32) + b3_r[...]

    return pl.pallas_call(
        body,
        grid=(n // R,),
        in_specs=[
            pl.BlockSpec((R, h), lambda i: (i, 0)),
            pl.BlockSpec((R, h), lambda i: (i, 0)),
            pl.BlockSpec((R, 2), lambda i: (i, 0)),
            pl.BlockSpec((h, h), lambda i: (0, 0)),
            pl.BlockSpec((h, h), lambda i: (0, 0)),
            pl.BlockSpec((1, h), lambda i: (0, 0)),
            pl.BlockSpec((h, cdim), lambda i: (0, 0)),
            pl.BlockSpec((h, cdim), lambda i: (0, 0)),
            pl.BlockSpec((1, cdim), lambda i: (0, 0)),
        ],
        out_specs=[
            pl.BlockSpec((R, cdim), lambda i: (i, 0)),
            pl.BlockSpec((R, cdim), lambda i: (i, 0)),
        ],
        out_shape=[
            jax.ShapeDtypeStruct((n, cdim), jnp.float32),
            jax.ShapeDtypeStruct((n, cdim), jnp.float32),
        ],
    )(h1, agg, deg2, Ws2, Wn2, b2.reshape(1, h), Ws3, Wn3, b3.reshape(1, cdim))


def _tc_final(q, agg_a, agg_b, deg2):
    n, cdim = q.shape
    R = 1000

    def body(q_r, aa_r, ab_r, deg_r, out_r):
        deg = deg_r[:, 0:1] + deg_r[:, 1:2]
        inv = 1.0 / jnp.maximum(deg, 1.0)
        out_r[...] = q_r[...] + (aa_r[...] + ab_r[...]) * inv

    return pl.pallas_call(
        body,
        grid=(n // R,),
        in_specs=[
            pl.BlockSpec((R, cdim), lambda i: (i, 0)),
            pl.BlockSpec((R, cdim), lambda i: (i, 0)),
            pl.BlockSpec((R, cdim), lambda i: (i, 0)),
            pl.BlockSpec((R, 2), lambda i: (i, 0)),
        ],
        out_specs=pl.BlockSpec((R, cdim), lambda i: (i, 0)),
        out_shape=jax.ShapeDtypeStruct((n, cdim), jnp.float32),
    )(q, agg_a, agg_b, deg2)


def _split_cols(x, n_pad):
    n, w = x.shape
    xp = jnp.pad(x, ((0, n_pad - n), (0, 0)))
    return (xp.reshape(n_pad, NC, w // NC).transpose(1, 0, 2)
            .reshape(NC * n_pad, w // NC))


def _unsplit_cols(y, n_pad, n, w):
    return (y.reshape(NC, n_pad, w // NC)[:, :n].transpose(1, 0, 2)
            .reshape(n, w))


def kernel(features, edge_index, Ws1, Wn1, b1, Ws2, Wn2, b2, Ws3, Wn3, b3):
    n, d = features.shape
    h = Ws1.shape[1]
    cdim = Ws3.shape[1]
    e = edge_index.shape[1]
    src = edge_index[0]
    dst = edge_index[1]
    align = NS * ZR
    n_pad = ((n + align - 1) // align) * align

    x_pad = jnp.pad(features, ((0, n_pad - n), (0, 0)))
    agg1, degf = _sc_aggregate(x_pad, src, dst, n_pad, e, d,
                               edge_split=True, with_deg=True)
    deg2 = jnp.concatenate(
        [degf[:n, 0:1], degf[n_pad:n_pad + n, 0:1]], axis=1)
    h1 = _tc_combine1(features, agg1[:n], agg1[n_pad:n_pad + n], deg2,
                      Ws1, Wn1, b1)

    agg2_2 = _sc_aggregate(_split_cols(h1, n_pad), src, dst, n_pad, e,
                           h // NC, edge_split=False, with_deg=False)[0]
    p3, q3 = _tc_combine2(h1, _unsplit_cols(agg2_2, n_pad, n, h), deg2,
                          Ws2, Wn2, b2, Ws3, Wn3, b3)

    p3_pad = jnp.pad(p3, ((0, n_pad - n), (0, 0)))
    agg3 = _sc_aggregate(p3_pad, src, dst, n_pad, e, cdim,
                         edge_split=True, with_deg=False)[0]
    return _tc_final(q3, agg3[:n], agg3[n_pad:n_pad + n], deg2)


# speedup vs baseline: 8.6085x; 1.7649x over previous
import functools

import jax
import jax.numpy as jnp
from jax import lax
from jax.experimental import pallas as pl
from jax.experimental.pallas import tpu as pltpu
from jax.experimental.pallas import tpu_sc as plsc

NC = 2
NS = 16
LANES = 16
ZR = 80


def _sc_aggregate(x2, src4, dst4, n_nodes, n_edges, wc, edge_split):
    ept = n_edges // (NC * NS if edge_split else NS)
    K = 40 if edge_split else 32
    NBUF = 5
    n_chunks = ept // K
    n_rounds = n_chunks // NBUF
    assert ept % K == 0 and n_chunks % NBUF == 0
    rpt = n_nodes // NS
    assert n_nodes % NS == 0 and rpt % K == 0

    mesh = plsc.VectorSubcoreMesh(core_axis_name="c", subcore_axis_name="s")

    scratch = (
        pltpu.VMEM_SHARED((n_nodes, wc), jnp.float32),
        pltpu.VMEM((NBUF, K), jnp.int32),
        pltpu.VMEM((NBUF, K), jnp.int32),
        [pltpu.VMEM((K, wc), jnp.float32) for _ in range(NBUF)],
        [pltpu.SemaphoreType.DMA for _ in range(NBUF)],
        [pltpu.SemaphoreType.DMA for _ in range(NBUF)],
        pltpu.SemaphoreType.DMA,
        pltpu.SemaphoreType.DMA,
    )

    @functools.partial(
        pl.kernel,
        out_type=jax.ShapeDtypeStruct((NC * n_nodes, wc), jnp.float32),
        mesh=mesh, scratch_types=scratch,
        compiler_params=pltpu.CompilerParams(use_tc_tiling_on_sc=False))
    def agg_kernel(src_h, dst_h, x_h, out_h,
                   acc, sidx_v, didx_v, rows, gsems, ssems, sisem, disem):
        c = lax.axis_index("c")
        s = lax.axis_index("s")
        wid = (c * NS + s) if edge_split else s
        off = c * n_nodes

        zvec = jnp.zeros((LANES,), jnp.float32)

        def fill_zero(r, _):
            for k in range(wc // LANES):
                rows[0][r, pl.ds(k * LANES, LANES)] = zvec
            return 0
        lax.fori_loop(0, K, fill_zero, 0)

        def zero_acc(j, _):
            pltpu.sync_copy(rows[0], acc.at[pl.ds(s * rpt + j * K, K)])
            return 0
        lax.fori_loop(0, rpt // K, zero_acc, 0)

        plsc.subcore_barrier()

        def idx_start(r):
            pltpu.async_copy(src_h.at[wid, r], sidx_v, sisem)
            pltpu.async_copy(dst_h.at[wid, r], didx_v, disem)

        def idx_wait():
            pltpu.make_async_copy(src_h.at[wid, 0], sidx_v, sisem).wait()
            pltpu.make_async_copy(dst_h.at[wid, 0], didx_v, disem).wait()
            if not edge_split:
                for b in range(NBUF):
                    for j in range(K // LANES):
                        sl = pl.ds(j * LANES, LANES)
                        sidx_v[b, sl] = sidx_v[b, sl] + off

        def gather_start(b):
            pltpu.async_copy(x_h.at[sidx_v.at[b]], rows[b], gsems[b])

        def gather_wait(b):
            pltpu.make_async_copy(x_h.at[sidx_v.at[0]], rows[b],
                                  gsems[b]).wait()

        def scatter_start(b):
            pltpu.async_copy(rows[b], acc.at[didx_v.at[b]], ssems[b],
                             add=True)

        def scatter_wait(b):
            pltpu.make_async_copy(rows[b], acc.at[didx_v.at[0]],
                                  ssems[b]).wait()

        idx_start(0)
        idx_wait()
        for b in range(NBUF):
            gather_start(b)

        def round_body(r, _):
            for b in range(NBUF):
                gather_wait(b)
                scatter_start(b)
            idx_start(r + 1)
            idx_wait()
            for b in range(NBUF):
                scatter_wait(b)
                gather_start(b)
            return 0
        lax.fori_loop(0, n_rounds - 1, round_body, 0)

        for b in range(NBUF):
            gather_wait(b)
            scatter_start(b)
        for b in range(NBUF):
            scatter_wait(b)

        plsc.subcore_barrier()

        r0 = off + s * rpt
        pltpu.sync_copy(acc.at[pl.ds(s * rpt, rpt)], out_h.at[pl.ds(r0, rpt)])

    return agg_kernel(src4, dst4, x2)


def _tc_combine1(x, agg_a, agg_b, deg2, Ws, Wn, b):
    n, d = x.shape
    h = Ws.shape[1]
    R = 1000

    def body(x_r, aa_r, ab_r, deg_r, Ws_r, Wn_r, b_r, out_r):
        deg = deg_r[:, 0:1] + deg_r[:, 1:2]
        inv = 1.0 / jnp.maximum(deg, 1.0)
        hn = (aa_r[...] + ab_r[...]) * inv
        acc = jnp.dot(x_r[...], Ws_r[...], preferred_element_type=jnp.float32)
        acc = acc + jnp.dot(hn, Wn_r[...], preferred_element_type=jnp.float32)
        out_r[...] = jnp.maximum(acc + b_r[...], 0.0)

    return pl.pallas_call(
        body,
        grid=(n // R,),
        in_specs=[
            pl.BlockSpec((R, d), lambda i: (i, 0)),
            pl.BlockSpec((R, d), lambda i: (i, 0)),
            pl.BlockSpec((R, d), lambda i: (i, 0)),
            pl.BlockSpec((R, 2), lambda i: (i, 0)),
            pl.BlockSpec((d, h), lambda i: (0, 0)),
            pl.BlockSpec((d, h), lambda i: (0, 0)),
            pl.BlockSpec((1, h), lambda i: (0, 0)),
        ],
        out_specs=pl.BlockSpec((R, h), lambda i: (i, 0)),
        out_shape=jax.ShapeDtypeStruct((n, h), jnp.float32),
    )(x, agg_a, agg_b, deg2, Ws, Wn, b.reshape(1, h))


def _tc_combine2(h1, agg, deg2, Ws2, Wn2, b2, Ws3, Wn3, b3):
    n, h = h1.shape
    cdim = Ws3.shape[1]
    R = 1000

    def body(h1_r, agg_r, deg_r, Ws2_r, Wn2_r, b2_r, Ws3_r, Wn3_r, b3_r,
             p_r, q_r):
        deg = deg_r[:, 0:1] + deg_r[:, 1:2]
        inv = 1.0 / jnp.maximum(deg, 1.0)
        hn = agg_r[...] * inv
        acc = jnp.dot(h1_r[...], Ws2_r[...], preferred_element_type=jnp.float32)
        acc = acc + jnp.dot(hn, Wn2_r[...], preferred_element_type=jnp.float32)
        h2 = jnp.maximum(acc + b2_r[...], 0.0)
        p_r[...] = jnp.dot(h2, Wn3_r[...], preferred_element_type=jnp.float32)
        q_r[...] = jnp.dot(h2, Ws3_r[...],
                           preferred_element_type=jnp.float32) + b3_r[...]

    return pl.pallas_call(
        body,
        grid=(n // R,),
        in_specs=[
            pl.BlockSpec((R, h), lambda i: (i, 0)),
            pl.BlockSpec((R, h), lambda i: (i, 0)),
            pl.BlockSpec((R, 2), lambda i: (i, 0)),
            pl.BlockSpec((h, h), lambda i: (0, 0)),
            pl.BlockSpec((h, h), lambda i: (0, 0)),
            pl.BlockSpec((1, h), lambda i: (0, 0)),
            pl.BlockSpec((h, cdim), lambda i: (0, 0)),
            pl.BlockSpec((h, cdim), lambda i: (0, 0)),
            pl.BlockSpec((1, cdim), lambda i: (0, 0)),
        ],
        out_specs=[
            pl.BlockSpec((R, cdim), lambda i: (i, 0)),
            pl.BlockSpec((R, cdim), lambda i: (i, 0)),
        ],
        out_shape=[
            jax.ShapeDtypeStruct((n, cdim), jnp.float32),
            jax.ShapeDtypeStruct((n, cdim), jnp.float32),
        ],
    )(h1, agg, deg2, Ws2, Wn2, b2.reshape(1, h), Ws3, Wn3, b3.reshape(1, cdim))


def _tc_final(q, agg_a, agg_b, deg2):
    n, cdim = q.shape
    R = 1000

    def body(q_r, aa_r, ab_r, deg_r, out_r):
        deg = deg_r[:, 0:1] + deg_r[:, 1:2]
        inv = 1.0 / jnp.maximum(deg, 1.0)
        out_r[...] = q_r[...] + (aa_r[...] + ab_r[...]) * inv

    return pl.pallas_call(
        body,
        grid=(n // R,),
        in_specs=[
            pl.BlockSpec((R, cdim), lambda i: (i, 0)),
            pl.BlockSpec((R, cdim), lambda i: (i, 0)),
            pl.BlockSpec((R, cdim), lambda i: (i, 0)),
            pl.BlockSpec((R, 2), lambda i: (i, 0)),
        ],
        out_specs=pl.BlockSpec((R, cdim), lambda i: (i, 0)),
        out_shape=jax.ShapeDtypeStruct((n, cdim), jnp.float32),
    )(q, agg_a, agg_b, deg2)


def _split_cols(x, n_pad):
    n, w = x.shape
    xp = jnp.pad(x, ((0, n_pad - n), (0, 0)))
    return (xp.reshape(n_pad, NC, w // NC).transpose(1, 0, 2)
            .reshape(NC * n_pad, w // NC))


def _unsplit_cols(y, n_pad, n, w):
    return (y.reshape(NC, n_pad, w // NC)[:, :n].transpose(1, 0, 2)
            .reshape(n, w))


def kernel(features, edge_index, Ws1, Wn1, b1, Ws2, Wn2, b2, Ws3, Wn3, b3):
    n, d = features.shape
    h = Ws1.shape[1]
    cdim = Ws3.shape[1]
    e = edge_index.shape[1]
    src = edge_index[0]
    dst = edge_index[1]
    align = NS * ZR
    n_pad = ((n + align - 1) // align) * align
    KE, KC, NBUF = 40, 32, 5
    src_e = src.reshape(NC * NS, e // (NC * NS * NBUF * KE), NBUF, KE)
    dst_e = dst.reshape(NC * NS, e // (NC * NS * NBUF * KE), NBUF, KE)
    src_c = src.reshape(NS, e // (NS * NBUF * KC), NBUF, KC)
    dst_c = dst.reshape(NS, e // (NS * NBUF * KC), NBUF, KC)

    degf = _sc_aggregate(jnp.ones((n_pad, cdim), jnp.float32),
                         src_e, dst_e, n_pad, e, cdim, edge_split=True)
    deg2 = jnp.concatenate(
        [degf[:n, 0:1], degf[n_pad:n_pad + n, 0:1]], axis=1)

    x_pad = jnp.pad(features, ((0, n_pad - n), (0, 0)))
    agg1 = _sc_aggregate(x_pad, src_e, dst_e, n_pad, e, d, edge_split=True)
    h1 = _tc_combine1(features, agg1[:n], agg1[n_pad:n_pad + n], deg2,
                      Ws1, Wn1, b1)

    agg2_2 = _sc_aggregate(_split_cols(h1, n_pad), src_c, dst_c, n_pad, e,
                           h // NC, edge_split=False)
    p3, q3 = _tc_combine2(h1, _unsplit_cols(agg2_2, n_pad, n, h), deg2,
                          Ws2, Wn2, b2, Ws3, Wn3, b3)

    p3_pad = jnp.pad(p3, ((0, n_pad - n), (0, 0)))
    agg3 = _sc_aggregate(p3_pad, src_e, dst_e, n_pad, e, cdim,
                         edge_split=True)
    return _tc_final(q3, agg3[:n], agg3[n_pad:n_pad + n], deg2)


# speedup vs baseline: 9.1851x; 1.0670x over previous
import functools

import jax
import jax.numpy as jnp
from jax import lax
from jax.experimental import pallas as pl
from jax.experimental.pallas import tpu as pltpu
from jax.experimental.pallas import tpu_sc as plsc

NC = 2
NS = 16
LANES = 16
ZR = 80


def _sc_aggregate(x2, src4, dst4, n_nodes, n_edges, wc, edge_split):
    ept = n_edges // (NC * NS if edge_split else NS)
    K = 40 if edge_split else 32
    NBUF = 5
    n_chunks = ept // K
    n_rounds = n_chunks // NBUF
    assert ept % K == 0 and n_chunks % NBUF == 0
    rpt = n_nodes // NS
    assert n_nodes % NS == 0 and rpt % K == 0

    mesh = plsc.VectorSubcoreMesh(core_axis_name="c", subcore_axis_name="s")

    scratch = (
        pltpu.VMEM_SHARED((n_nodes, wc), jnp.float32),
        pltpu.VMEM((NBUF, K), jnp.int32),
        pltpu.VMEM((NBUF, K), jnp.int32),
        [pltpu.VMEM((K, wc), jnp.float32) for _ in range(NBUF)],
        [pltpu.SemaphoreType.DMA for _ in range(NBUF)],
        [pltpu.SemaphoreType.DMA for _ in range(NBUF)],
        pltpu.SemaphoreType.DMA,
        pltpu.SemaphoreType.DMA,
    )

    @functools.partial(
        pl.kernel,
        out_type=jax.ShapeDtypeStruct((NC * n_nodes, wc), jnp.float32),
        mesh=mesh, scratch_types=scratch,
        compiler_params=pltpu.CompilerParams(use_tc_tiling_on_sc=False))
    def agg_kernel(src_h, dst_h, x_h, out_h,
                   acc, sidx_v, didx_v, rows, gsems, ssems, sisem, disem):
        c = lax.axis_index("c")
        s = lax.axis_index("s")
        wid = (c * NS + s) if edge_split else s
        off = c * n_nodes

        zvec = jnp.zeros((LANES,), jnp.float32)

        def fill_zero(r, _):
            for k in range(wc // LANES):
                rows[0][r, pl.ds(k * LANES, LANES)] = zvec
            return 0
        lax.fori_loop(0, K, fill_zero, 0)

        def zero_acc(j, _):
            pltpu.sync_copy(rows[0], acc.at[pl.ds(s * rpt + j * K, K)])
            return 0
        lax.fori_loop(0, rpt // K, zero_acc, 0)

        plsc.subcore_barrier()

        def idx_start(r):
            pltpu.async_copy(src_h.at[wid, r], sidx_v, sisem)
            pltpu.async_copy(dst_h.at[wid, r], didx_v, disem)

        def idx_wait():
            pltpu.make_async_copy(src_h.at[wid, 0], sidx_v, sisem).wait()
            pltpu.make_async_copy(dst_h.at[wid, 0], didx_v, disem).wait()
            if not edge_split:
                for b in range(NBUF):
                    for j in range(K // LANES):
                        sl = pl.ds(j * LANES, LANES)
                        sidx_v[b, sl] = sidx_v[b, sl] + off

        def gather_start(b):
            pltpu.async_copy(x_h.at[sidx_v.at[b]], rows[b], gsems[b])

        def gather_wait(b):
            pltpu.make_async_copy(x_h.at[sidx_v.at[0]], rows[b],
                                  gsems[b]).wait()

        def scatter_start(b):
            pltpu.async_copy(rows[b], acc.at[didx_v.at[b]], ssems[b],
                             add=True)

        def scatter_wait(b):
            pltpu.make_async_copy(rows[b], acc.at[didx_v.at[0]],
                                  ssems[b]).wait()

        idx_start(0)
        idx_wait()
        for b in range(NBUF):
            gather_start(b)

        def round_body(r, _):
            for b in range(NBUF):
                gather_wait(b)
                scatter_start(b)
            idx_start(r + 1)
            idx_wait()
            for b in range(NBUF):
                scatter_wait(b)
                gather_start(b)
            return 0
        lax.fori_loop(0, n_rounds - 1, round_body, 0)

        for b in range(NBUF):
            gather_wait(b)
            scatter_start(b)
        for b in range(NBUF):
            scatter_wait(b)

        plsc.subcore_barrier()

        r0 = off + s * rpt
        pltpu.sync_copy(acc.at[pl.ds(s * rpt, rpt)], out_h.at[pl.ds(r0, rpt)])

    return agg_kernel(src4, dst4, x2)


def _tc_combine1(x, agg_a, agg_b, deg2, Ws, Wn, b):
    n, d = x.shape
    h = Ws.shape[1]
    R = 1000

    def body(x_r, aa_r, ab_r, deg_r, Ws_r, Wn_r, b_r, out_r):
        deg = deg_r[:, 0:1] + deg_r[:, 1:2]
        inv = 1.0 / jnp.maximum(deg, 1.0)
        hn = (aa_r[...] + ab_r[...]) * inv
        acc = jnp.dot(x_r[...], Ws_r[...], preferred_element_type=jnp.float32)
        acc = acc + jnp.dot(hn, Wn_r[...], preferred_element_type=jnp.float32)
        out_r[...] = jnp.maximum(acc + b_r[...], 0.0)

    return pl.pallas_call(
        body,
        grid=(n // R,),
        in_specs=[
            pl.BlockSpec((R, d), lambda i: (i, 0)),
            pl.BlockSpec((R, d), lambda i: (i, 0)),
            pl.BlockSpec((R, d), lambda i: (i, 0)),
            pl.BlockSpec((R, 2), lambda i: (i, 0)),
            pl.BlockSpec((d, h), lambda i: (0, 0)),
            pl.BlockSpec((d, h), lambda i: (0, 0)),
            pl.BlockSpec((1, h), lambda i: (0, 0)),
        ],
        out_specs=pl.BlockSpec((R, h), lambda i: (i, 0)),
        out_shape=jax.ShapeDtypeStruct((n, h), jnp.float32),
    )(x, agg_a, agg_b, deg2, Ws, Wn, b.reshape(1, h))


def _tc_combine2(h1, agg, deg2, Ws2, Wn2, b2, Ws3, Wn3, b3):
    n, h = h1.shape
    cdim = Ws3.shape[1]
    R = 1000

    def body(h1_r, agg_r, deg_r, Ws2_r, Wn2_r, b2_r, Ws3_r, Wn3_r, b3_r,
             p_r, q_r):
        deg = deg_r[:, 0:1] + deg_r[:, 1:2]
        inv = 1.0 / jnp.maximum(deg, 1.0)
        hn = agg_r[...] * inv
        acc = jnp.dot(h1_r[...], Ws2_r[...], preferred_element_type=jnp.float32)
        acc = acc + jnp.dot(hn, Wn2_r[...], preferred_element_type=jnp.float32)
        h2 = jnp.maximum(acc + b2_r[...], 0.0)
        p_r[...] = jnp.dot(h2, Wn3_r[...], preferred_element_type=jnp.float32)
        q_r[...] = jnp.dot(h2, Ws3_r[...],
                           preferred_element_type=jnp.float32) + b3_r[...]

    return pl.pallas_call(
        body,
        grid=(n // R,),
        in_specs=[
            pl.BlockSpec((R, h), lambda i: (i, 0)),
            pl.BlockSpec((R, h), lambda i: (i, 0)),
            pl.BlockSpec((R, 2), lambda i: (i, 0)),
            pl.BlockSpec((h, h), lambda i: (0, 0)),
            pl.BlockSpec((h, h), lambda i: (0, 0)),
            pl.BlockSpec((1, h), lambda i: (0, 0)),
            pl.BlockSpec((h, cdim), lambda i: (0, 0)),
            pl.BlockSpec((h, cdim), lambda i: (0, 0)),
            pl.BlockSpec((1, cdim), lambda i: (0, 0)),
        ],
        out_specs=[
            pl.BlockSpec((R, cdim), lambda i: (i, 0)),
            pl.BlockSpec((R, cdim), lambda i: (i, 0)),
        ],
        out_shape=[
            jax.ShapeDtypeStruct((n, cdim), jnp.float32),
            jax.ShapeDtypeStruct((n, cdim), jnp.float32),
        ],
    )(h1, agg, deg2, Ws2, Wn2, b2.reshape(1, h), Ws3, Wn3, b3.reshape(1, cdim))


def _tc_final(q, agg_a, agg_b, deg2):
    n, cdim = q.shape
    R = 1000

    def body(q_r, aa_r, ab_r, deg_r, out_r):
        deg = deg_r[:, 0:1] + deg_r[:, 1:2]
        inv = 1.0 / jnp.maximum(deg, 1.0)
        out_r[...] = q_r[...] + (aa_r[...] + ab_r[...]) * inv

    return pl.pallas_call(
        body,
        grid=(n // R,),
        in_specs=[
            pl.BlockSpec((R, cdim), lambda i: (i, 0)),
            pl.BlockSpec((R, cdim), lambda i: (i, 0)),
            pl.BlockSpec((R, cdim), lambda i: (i, 0)),
            pl.BlockSpec((R, 2), lambda i: (i, 0)),
        ],
        out_specs=pl.BlockSpec((R, cdim), lambda i: (i, 0)),
        out_shape=jax.ShapeDtypeStruct((n, cdim), jnp.float32),
    )(q, agg_a, agg_b, deg2)


def _split_cols(x, n_pad):
    n, w = x.shape
    xp = jnp.pad(x, ((0, n_pad - n), (0, 0)))
    return (xp.reshape(n_pad, NC, w // NC).transpose(1, 0, 2)
            .reshape(NC * n_pad, w // NC))


def _unsplit_cols(y, n_pad, n, w):
    return (y.reshape(NC, n_pad, w // NC)[:, :n].transpose(1, 0, 2)
            .reshape(n, w))


def kernel(features, edge_index, Ws1, Wn1, b1, Ws2, Wn2, b2, Ws3, Wn3, b3):
    n, d = features.shape
    h = Ws1.shape[1]
    cdim = Ws3.shape[1]
    e = edge_index.shape[1]
    src = edge_index[0]
    dst = edge_index[1]
    align = NS * ZR
    n_pad = ((n + align - 1) // align) * align
    KE, KC, NBUF = 40, 32, 5
    src_e = src.reshape(NC * NS, e // (NC * NS * NBUF * KE), NBUF, KE)
    dst_e = dst.reshape(NC * NS, e // (NC * NS * NBUF * KE), NBUF, KE)
    src_c = src.reshape(NS, e // (NS * NBUF * KC), NBUF, KC)
    dst_c = dst.reshape(NS, e // (NS * NBUF * KC), NBUF, KC)

    x_aug = jnp.concatenate(
        [jnp.pad(features, ((0, n_pad - n), (0, 0))),
         jnp.ones((n_pad, LANES), jnp.float32)], axis=1)
    agg1 = _sc_aggregate(x_aug, src_e, dst_e, n_pad, e, d + LANES,
                         edge_split=True)
    deg2 = jnp.concatenate(
        [agg1[:n, d:d + 1], agg1[n_pad:n_pad + n, d:d + 1]], axis=1)
    h1 = _tc_combine1(features, agg1[:n, :d], agg1[n_pad:n_pad + n, :d],
                      deg2, Ws1, Wn1, b1)

    agg2_2 = _sc_aggregate(_split_cols(h1, n_pad), src_c, dst_c, n_pad, e,
                           h // NC, edge_split=False)
    p3, q3 = _tc_combine2(h1, _unsplit_cols(agg2_2, n_pad, n, h), deg2,
                          Ws2, Wn2, b2, Ws3, Wn3, b3)

    p3_pad = jnp.pad(p3, ((0, n_pad - n), (0, 0)))
    agg3 = _sc_aggregate(p3_pad, src_e, dst_e, n_pad, e, cdim,
                         edge_split=True)
    return _tc_final(q3, agg3[:n], agg3[n_pad:n_pad + n], deg2)


# speedup vs baseline: 9.9497x; 1.0832x over previous
import functools

import jax
import jax.numpy as jnp
from jax import lax
from jax.experimental import pallas as pl
from jax.experimental.pallas import tpu as pltpu
from jax.experimental.pallas import tpu_sc as plsc

NC = 2
NS = 16
LANES = 16
ZR = 80


def _sc_aggregate(x2, src4, dst4, n_nodes, n_edges, wc, edge_split):
    ept = n_edges // (NC * NS if edge_split else NS)
    K = 40 if edge_split else 32
    NBUF = 5
    n_chunks = ept // K
    n_rounds = n_chunks // NBUF
    assert ept % K == 0 and n_chunks % NBUF == 0
    rpt = n_nodes // NS
    assert n_nodes % NS == 0 and rpt % K == 0

    mesh = plsc.VectorSubcoreMesh(core_axis_name="c", subcore_axis_name="s")

    scratch = (
        pltpu.VMEM_SHARED((n_nodes, wc), jnp.float32),
        pltpu.VMEM((NBUF, K), jnp.int32),
        pltpu.VMEM((NBUF, K), jnp.int32),
        [pltpu.VMEM((K, wc), jnp.float32) for _ in range(NBUF)],
        [pltpu.SemaphoreType.DMA for _ in range(NBUF)],
        [pltpu.SemaphoreType.DMA for _ in range(NBUF)],
        pltpu.SemaphoreType.DMA,
        pltpu.SemaphoreType.DMA,
    )

    @functools.partial(
        pl.kernel,
        out_type=jax.ShapeDtypeStruct((NC * n_nodes, wc), jnp.float32),
        mesh=mesh, scratch_types=scratch,
        compiler_params=pltpu.CompilerParams(use_tc_tiling_on_sc=False))
    def agg_kernel(src_h, dst_h, x_h, out_h,
                   acc, sidx_v, didx_v, rows, gsems, ssems, sisem, disem):
        c = lax.axis_index("c")
        s = lax.axis_index("s")
        wid = (c * NS + s) if edge_split else s
        off = c * n_nodes

        zvec = jnp.zeros((LANES,), jnp.float32)

        def fill_zero(r, _):
            for k in range(wc // LANES):
                rows[0][r, pl.ds(k * LANES, LANES)] = zvec
            return 0
        lax.fori_loop(0, K, fill_zero, 0)

        def zero_acc(j, _):
            pltpu.sync_copy(rows[0], acc.at[pl.ds(s * rpt + j * K, K)])
            return 0
        lax.fori_loop(0, rpt // K, zero_acc, 0)

        plsc.subcore_barrier()

        def idx_start(r):
            pltpu.async_copy(src_h.at[wid, r], sidx_v, sisem)
            pltpu.async_copy(dst_h.at[wid, r], didx_v, disem)

        def idx_wait():
            pltpu.make_async_copy(src_h.at[wid, 0], sidx_v, sisem).wait()
            pltpu.make_async_copy(dst_h.at[wid, 0], didx_v, disem).wait()
            if not edge_split:
                for b in range(NBUF):
                    for j in range(K // LANES):
                        sl = pl.ds(j * LANES, LANES)
                        sidx_v[b, sl] = sidx_v[b, sl] + off

        def gather_start(b):
            pltpu.async_copy(x_h.at[sidx_v.at[b]], rows[b], gsems[b])

        def gather_wait(b):
            pltpu.make_async_copy(x_h.at[sidx_v.at[0]], rows[b],
                                  gsems[b]).wait()

        def scatter_start(b):
            pltpu.async_copy(rows[b], acc.at[didx_v.at[b]], ssems[b],
                             add=True)

        def scatter_wait(b):
            pltpu.make_async_copy(rows[b], acc.at[didx_v.at[0]],
                                  ssems[b]).wait()

        idx_start(0)
        idx_wait()
        for b in range(NBUF):
            gather_start(b)

        def round_body(r, _):
            for b in range(NBUF):
                gather_wait(b)
                scatter_start(b)
            idx_start(r + 1)
            idx_wait()
            for b in range(NBUF):
                scatter_wait(b)
                gather_start(b)
            return 0
        lax.fori_loop(0, n_rounds - 1, round_body, 0)

        for b in range(NBUF):
            gather_wait(b)
            scatter_start(b)
        for b in range(NBUF):
            scatter_wait(b)

        plsc.subcore_barrier()

        r0 = off + s * rpt
        pltpu.sync_copy(acc.at[pl.ds(s * rpt, rpt)], out_h.at[pl.ds(r0, rpt)])

    return agg_kernel(src4, dst4, x2)


def _tc_combine1(x, agg1_3, Ws, Wn, b):
    n, d = x.shape
    h = Ws.shape[1]
    n_pad = agg1_3.shape[1]
    R = 1000

    def body(x_r, agg_r, Ws_r, Wn_r, b_r, out_r, spl_r):
        a = agg_r[0]
        bb = agg_r[1]
        deg = a[:, d:d + 1] + bb[:, d:d + 1]
        inv = 1.0 / jnp.maximum(deg, 1.0)
        hn = (a[:, :d] + bb[:, :d]) * inv
        acc = jnp.dot(x_r[...], Ws_r[...], preferred_element_type=jnp.float32)
        acc = acc + jnp.dot(hn, Wn_r[...], preferred_element_type=jnp.float32)
        out = jnp.maximum(acc + b_r[...], 0.0)
        out_r[...] = out
        spl_r[0] = out[:, :h // 2]
        spl_r[1] = out[:, h // 2:]

    return pl.pallas_call(
        body,
        grid=(n // R,),
        in_specs=[
            pl.BlockSpec((R, d), lambda i: (i, 0)),
            pl.BlockSpec((2, R, d + LANES), lambda i: (0, i, 0)),
            pl.BlockSpec((d, h), lambda i: (0, 0)),
            pl.BlockSpec((d, h), lambda i: (0, 0)),
            pl.BlockSpec((1, h), lambda i: (0, 0)),
        ],
        out_specs=[
            pl.BlockSpec((R, h), lambda i: (i, 0)),
            pl.BlockSpec((2, R, h // 2), lambda i: (0, i, 0)),
        ],
        out_shape=[
            jax.ShapeDtypeStruct((n, h), jnp.float32),
            jax.ShapeDtypeStruct((2, n_pad, h // 2), jnp.float32),
        ],
    )(x, agg1_3, Ws, Wn, b.reshape(1, h))


def _tc_combine2(h1, agg2_3, deg2, Ws2, Wn2, b2, Ws3, Wn3, b3, n_pad):
    n, h = h1.shape
    cdim = Ws3.shape[1]
    R = 1000

    def body(h1_r, agg_r, deg_r, Ws2_r, Wn2_r, b2_r, Ws3_r, Wn3_r, b3_r,
             p_r, q_r):
        deg = deg_r[:, 0:1] + deg_r[:, 1:2]
        inv = 1.0 / jnp.maximum(deg, 1.0)
        hn = jnp.concatenate([agg_r[0], agg_r[1]], axis=1) * inv
        acc = jnp.dot(h1_r[...], Ws2_r[...], preferred_element_type=jnp.float32)
        acc = acc + jnp.dot(hn, Wn2_r[...], preferred_element_type=jnp.float32)
        h2 = jnp.maximum(acc + b2_r[...], 0.0)
        p_r[...] = jnp.dot(h2, Wn3_r[...], preferred_element_type=jnp.float32)
        q_r[...] = jnp.dot(h2, Ws3_r[...],
                           preferred_element_type=jnp.float32) + b3_r[...]

    return pl.pallas_call(
        body,
        grid=(n // R,),
        in_specs=[
            pl.BlockSpec((R, h), lambda i: (i, 0)),
            pl.BlockSpec((2, R, h // 2), lambda i: (0, i, 0)),
            pl.BlockSpec((R, 2), lambda i: (i, 0)),
            pl.BlockSpec((h, h), lambda i: (0, 0)),
            pl.BlockSpec((h, h), lambda i: (0, 0)),
            pl.BlockSpec((1, h), lambda i: (0, 0)),
            pl.BlockSpec((h, cdim), lambda i: (0, 0)),
            pl.BlockSpec((h, cdim), lambda i: (0, 0)),
            pl.BlockSpec((1, cdim), lambda i: (0, 0)),
        ],
        out_specs=[
            pl.BlockSpec((R, cdim), lambda i: (i, 0)),
            pl.BlockSpec((R, cdim), lambda i: (i, 0)),
        ],
        out_shape=[
            jax.ShapeDtypeStruct((n_pad, cdim), jnp.float32),
            jax.ShapeDtypeStruct((n, cdim), jnp.float32),
        ],
    )(h1, agg2_3, deg2, Ws2, Wn2, b2.reshape(1, h), Ws3, Wn3,
      b3.reshape(1, cdim))


def _tc_final(q, agg3_3, deg2):
    n, cdim = q.shape
    R = 1000

    def body(q_r, agg_r, deg_r, out_r):
        deg = deg_r[:, 0:1] + deg_r[:, 1:2]
        inv = 1.0 / jnp.maximum(deg, 1.0)
        out_r[...] = q_r[...] + (agg_r[0] + agg_r[1]) * inv

    return pl.pallas_call(
        body,
        grid=(n // R,),
        in_specs=[
            pl.BlockSpec((R, cdim), lambda i: (i, 0)),
            pl.BlockSpec((2, R, cdim), lambda i: (0, i, 0)),
            pl.BlockSpec((R, 2), lambda i: (i, 0)),
        ],
        out_specs=pl.BlockSpec((R, cdim), lambda i: (i, 0)),
        out_shape=jax.ShapeDtypeStruct((n, cdim), jnp.float32),
    )(q, agg3_3, deg2)


def kernel(features, edge_index, Ws1, Wn1, b1, Ws2, Wn2, b2, Ws3, Wn3, b3):
    n, d = features.shape
    h = Ws1.shape[1]
    cdim = Ws3.shape[1]
    e = edge_index.shape[1]
    src = edge_index[0]
    dst = edge_index[1]
    align = NS * ZR
    n_pad = ((n + align - 1) // align) * align
    KE, KC, NBUF = 40, 32, 5
    src_e = src.reshape(NC * NS, e // (NC * NS * NBUF * KE), NBUF, KE)
    dst_e = dst.reshape(NC * NS, e // (NC * NS * NBUF * KE), NBUF, KE)
    src_c = src.reshape(NS, e // (NS * NBUF * KC), NBUF, KC)
    dst_c = dst.reshape(NS, e // (NS * NBUF * KC), NBUF, KC)

    x_aug = jnp.concatenate(
        [jnp.pad(features, ((0, n_pad - n), (0, 0))),
         jnp.ones((n_pad, LANES), jnp.float32)], axis=1)
    agg1 = _sc_aggregate(x_aug, src_e, dst_e, n_pad, e, d + LANES,
                         edge_split=True)
    agg1_3 = agg1.reshape(NC, n_pad, d + LANES)
    deg2 = jnp.concatenate(
        [agg1[:n, d:d + 1], agg1[n_pad:n_pad + n, d:d + 1]], axis=1)
    h1, h1s = _tc_combine1(features, agg1_3, Ws1, Wn1, b1)

    agg2_2 = _sc_aggregate(h1s.reshape(NC * n_pad, h // NC), src_c, dst_c,
                           n_pad, e, h // NC, edge_split=False)
    p3_pad, q3 = _tc_combine2(h1, agg2_2.reshape(NC, n_pad, h // NC), deg2,
                              Ws2, Wn2, b2, Ws3, Wn3, b3, n_pad)

    agg3 = _sc_aggregate(p3_pad, src_e, dst_e, n_pad, e, cdim,
                         edge_split=True)
    return _tc_final(q3, agg3.reshape(NC, n_pad, cdim), deg2)


# speedup vs baseline: 11.6618x; 1.1721x over previous
import functools

import jax
import jax.numpy as jnp
from jax import lax
from jax.experimental import pallas as pl
from jax.experimental.pallas import tpu as pltpu
from jax.experimental.pallas import tpu_sc as plsc

NC = 2
NS = 16
LANES = 16
ZR = 80


def _sc_aggregate(x2, src5, dst5, n_nodes, n_edges, wc, edge_split):
    K = src5.shape[4]
    NBUF = src5.shape[3]
    n_rounds = src5.shape[2]
    rpt = n_nodes // NS
    assert n_nodes % NS == 0 and rpt % K == 0

    mesh = plsc.VectorSubcoreMesh(core_axis_name="c", subcore_axis_name="s")

    scratch = (
        pltpu.VMEM_SHARED((n_nodes, wc), jnp.float32),
        pltpu.VMEM((2, NBUF, K), jnp.int32),
        pltpu.VMEM((2, NBUF, K), jnp.int32),
        [pltpu.VMEM((K, wc), jnp.float32) for _ in range(NBUF)],
        [pltpu.SemaphoreType.DMA for _ in range(NBUF)],
        [pltpu.SemaphoreType.DMA for _ in range(NBUF)],
        pltpu.SemaphoreType.DMA,
        pltpu.SemaphoreType.DMA,
    )

    @functools.partial(
        pl.kernel,
        out_type=jax.ShapeDtypeStruct((NC * n_nodes, wc), jnp.float32),
        mesh=mesh, scratch_types=scratch,
        compiler_params=pltpu.CompilerParams(use_tc_tiling_on_sc=False))
    def agg_kernel(src_h, dst_h, x_h, out_h,
                   acc, sidx_v, didx_v, rows, gsems, ssems, sisem, disem):
        c = lax.axis_index("c")
        s = lax.axis_index("s")

        zvec = jnp.zeros((LANES,), jnp.float32)

        def fill_zero(r, _):
            for k in range(wc // LANES):
                rows[0][r, pl.ds(k * LANES, LANES)] = zvec
            return 0
        lax.fori_loop(0, K, fill_zero, 0)

        def zero_acc(j, _):
            pltpu.sync_copy(rows[0], acc.at[pl.ds(s * rpt + j * K, K)])
            return 0
        lax.fori_loop(0, rpt // K, zero_acc, 0)

        plsc.subcore_barrier()

        def idx_start(r, par):
            pltpu.async_copy(src_h.at[c, s, r], sidx_v.at[par], sisem)
            pltpu.async_copy(dst_h.at[c, s, r], didx_v.at[par], disem)

        def idx_wait():
            pltpu.make_async_copy(src_h.at[c, s, 0], sidx_v.at[0],
                                  sisem).wait()
            pltpu.make_async_copy(dst_h.at[c, s, 0], didx_v.at[0],
                                  disem).wait()

        def gather_start(b, par):
            pltpu.async_copy(x_h.at[sidx_v.at[par, b]], rows[b], gsems[b])

        def gather_wait(b):
            pltpu.make_async_copy(x_h.at[sidx_v.at[0, 0]], rows[b],
                                  gsems[b]).wait()

        def scatter_start(b, par):
            pltpu.async_copy(rows[b], acc.at[didx_v.at[par, b]], ssems[b],
                             add=True)

        def scatter_wait(b):
            pltpu.make_async_copy(rows[b], acc.at[didx_v.at[0, 0]],
                                  ssems[b]).wait()

        idx_start(0, 0)
        idx_wait()
        for b in range(NBUF):
            gather_start(b, 0)
        idx_start(jnp.minimum(1, n_rounds - 1), 1)

        def round_body(r, _):
            par = lax.rem(r, 2)
            nxt = 1 - par
            for b in range(NBUF):
                gather_wait(b)
                scatter_start(b, par)
            idx_wait()
            for b in range(NBUF):
                scatter_wait(b)
                gather_start(b, nxt)
            idx_start(jnp.minimum(r + 2, n_rounds - 1), par)
            return 0
        lax.fori_loop(0, n_rounds - 1, round_body, 0)

        lpar = (n_rounds - 1) % 2
        for b in range(NBUF):
            gather_wait(b)
            scatter_start(b, lpar)
        idx_wait()
        for b in range(NBUF):
            scatter_wait(b)

        plsc.subcore_barrier()

        r0 = c * n_nodes + s * rpt
        pltpu.sync_copy(acc.at[pl.ds(s * rpt, rpt)], out_h.at[pl.ds(r0, rpt)])

    return agg_kernel(src5, dst5, x2)


def _tc_combine1(x, agg1_3, Ws, Wn, b):
    n, d = x.shape
    h = Ws.shape[1]
    n_pad = agg1_3.shape[1]
    R = 1000

    def body(x_r, agg_r, Ws_r, Wn_r, b_r, out_r, spl_r):
        a = agg_r[0]
        bb = agg_r[1]
        deg = a[:, d:d + 1] + bb[:, d:d + 1]
        inv = 1.0 / jnp.maximum(deg, 1.0)
        hn = (a[:, :d] + bb[:, :d]) * inv
        acc = jnp.dot(x_r[...], Ws_r[...], preferred_element_type=jnp.float32)
        acc = acc + jnp.dot(hn, Wn_r[...], preferred_element_type=jnp.float32)
        out = jnp.maximum(acc + b_r[...], 0.0)
        out_r[...] = out
        spl_r[0] = out[:, :h // 2]
        spl_r[1] = out[:, h // 2:]

    return pl.pallas_call(
        body,
        grid=(n // R,),
        in_specs=[
            pl.BlockSpec((R, d), lambda i: (i, 0)),
            pl.BlockSpec((2, R, d + LANES), lambda i: (0, i, 0)),
            pl.BlockSpec((d, h), lambda i: (0, 0)),
            pl.BlockSpec((d, h), lambda i: (0, 0)),
            pl.BlockSpec((1, h), lambda i: (0, 0)),
        ],
        out_specs=[
            pl.BlockSpec((R, h), lambda i: (i, 0)),
            pl.BlockSpec((2, R, h // 2), lambda i: (0, i, 0)),
        ],
        out_shape=[
            jax.ShapeDtypeStruct((n, h), jnp.float32),
            jax.ShapeDtypeStruct((2, n_pad, h // 2), jnp.float32),
        ],
    )(x, agg1_3, Ws, Wn, b.reshape(1, h))


def _tc_combine2(h1, agg2_3, deg2, Ws2, Wn2, b2, Ws3, Wn3, b3, n_pad):
    n, h = h1.shape
    cdim = Ws3.shape[1]
    R = 1000

    def body(h1_r, agg_r, deg_r, Ws2_r, Wn2_r, b2_r, Ws3_r, Wn3_r, b3_r,
             p_r, q_r):
        deg = deg_r[:, 0:1] + deg_r[:, 1:2]
        inv = 1.0 / jnp.maximum(deg, 1.0)
        hn = jnp.concatenate([agg_r[0], agg_r[1]], axis=1) * inv
        acc = jnp.dot(h1_r[...], Ws2_r[...], preferred_element_type=jnp.float32)
        acc = acc + jnp.dot(hn, Wn2_r[...], preferred_element_type=jnp.float32)
        h2 = jnp.maximum(acc + b2_r[...], 0.0)
        p_r[...] = jnp.dot(h2, Wn3_r[...], preferred_element_type=jnp.float32)
        q_r[...] = jnp.dot(h2, Ws3_r[...],
                           preferred_element_type=jnp.float32) + b3_r[...]

    return pl.pallas_call(
        body,
        grid=(n // R,),
        in_specs=[
            pl.BlockSpec((R, h), lambda i: (i, 0)),
            pl.BlockSpec((2, R, h // 2), lambda i: (0, i, 0)),
            pl.BlockSpec((R, 2), lambda i: (i, 0)),
            pl.BlockSpec((h, h), lambda i: (0, 0)),
            pl.BlockSpec((h, h), lambda i: (0, 0)),
            pl.BlockSpec((1, h), lambda i: (0, 0)),
            pl.BlockSpec((h, cdim), lambda i: (0, 0)),
            pl.BlockSpec((h, cdim), lambda i: (0, 0)),
            pl.BlockSpec((1, cdim), lambda i: (0, 0)),
        ],
        out_specs=[
            pl.BlockSpec((R, cdim), lambda i: (i, 0)),
            pl.BlockSpec((R, cdim), lambda i: (i, 0)),
        ],
        out_shape=[
            jax.ShapeDtypeStruct((n_pad, cdim), jnp.float32),
            jax.ShapeDtypeStruct((n, cdim), jnp.float32),
        ],
    )(h1, agg2_3, deg2, Ws2, Wn2, b2.reshape(1, h), Ws3, Wn3,
      b3.reshape(1, cdim))


def _tc_final(q, agg3_3, deg2):
    n, cdim = q.shape
    R = 1000

    def body(q_r, agg_r, deg_r, out_r):
        deg = deg_r[:, 0:1] + deg_r[:, 1:2]
        inv = 1.0 / jnp.maximum(deg, 1.0)
        out_r[...] = q_r[...] + (agg_r[0] + agg_r[1]) * inv

    return pl.pallas_call(
        body,
        grid=(n // R,),
        in_specs=[
            pl.BlockSpec((R, cdim), lambda i: (i, 0)),
            pl.BlockSpec((2, R, cdim), lambda i: (0, i, 0)),
            pl.BlockSpec((R, 2), lambda i: (i, 0)),
        ],
        out_specs=pl.BlockSpec((R, cdim), lambda i: (i, 0)),
        out_shape=jax.ShapeDtypeStruct((n, cdim), jnp.float32),
    )(q, agg3_3, deg2)


def kernel(features, edge_index, Ws1, Wn1, b1, Ws2, Wn2, b2, Ws3, Wn3, b3):
    n, d = features.shape
    h = Ws1.shape[1]
    cdim = Ws3.shape[1]
    e = edge_index.shape[1]
    src = edge_index[0]
    dst = edge_index[1]
    align = NS * ZR
    n_pad = ((n + align - 1) // align) * align
    KE, KC, NBUF = 40, 32, 5
    src_e = src.reshape(NC, NS, e // (NC * NS * NBUF * KE), NBUF, KE)
    dst_e = dst.reshape(NC, NS, e // (NC * NS * NBUF * KE), NBUF, KE)
    off = (jnp.arange(NC, dtype=jnp.int32) * n_pad)[:, None]
    src_c = (src[None, :] + off).reshape(
        NC, NS, e // (NS * NBUF * KC), NBUF, KC)
    dst_c = jnp.broadcast_to(dst, (NC, e)).reshape(
        NC, NS, e // (NS * NBUF * KC), NBUF, KC)

    x_aug = jnp.concatenate(
        [jnp.pad(features, ((0, n_pad - n), (0, 0))),
         jnp.ones((n_pad, LANES), jnp.float32)], axis=1)
    agg1 = _sc_aggregate(x_aug, src_e, dst_e, n_pad, e, d + LANES,
                         edge_split=True)
    agg1_3 = agg1.reshape(NC, n_pad, d + LANES)
    deg2 = jnp.concatenate(
        [agg1[:n, d:d + 1], agg1[n_pad:n_pad + n, d:d + 1]], axis=1)
    h1, h1s = _tc_combine1(features, agg1_3, Ws1, Wn1, b1)

    agg2_2 = _sc_aggregate(h1s.reshape(NC * n_pad, h // NC), src_c, dst_c,
                           n_pad, e, h // NC, edge_split=False)
    p3_pad, q3 = _tc_combine2(h1, agg2_2.reshape(NC, n_pad, h // NC), deg2,
                              Ws2, Wn2, b2, Ws3, Wn3, b3, n_pad)

    agg3 = _sc_aggregate(p3_pad, src_e, dst_e, n_pad, e, cdim,
                         edge_split=True)
    return _tc_final(q3, agg3.reshape(NC, n_pad, cdim), deg2)
